# baseline (speedup 1.0000x reference)
import functools

import jax
import jax.numpy as jnp
from jax import lax
from jax.experimental import pallas as pl
from jax.experimental.pallas import tpu as pltpu
from jax.experimental.pallas import tpu_sc as plsc

B, F, D, V = 4096, 100, 64, 100000
KEEP = 3200
NC, NS = 2, 16
NW = NC * NS
BPW = B // NW
CHB = 4
FP = 128
FD = F * D

_MESH = plsc.VectorSubcoreMesh(
    core_axis_name="c", subcore_axis_name="s", num_cores=NC, num_subcores=NS)
_SC_PARAMS = pltpu.CompilerParams(
    use_tc_tiling_on_sc=False, needs_layout_passes=False)


@functools.partial(
    pl.kernel,
    out_type=jax.ShapeDtypeStruct((NW, FD), jnp.float32),
    mesh=_MESH,
    scratch_types=[
        pltpu.VMEM((BPW, FP), jnp.int32),
        pltpu.VMEM((CHB, FP, D), jnp.float32),
        pltpu.VMEM((CHB, FP, D), jnp.float32),
        pltpu.VMEM((FD,), jnp.float32),
        pltpu.VMEM((BPW,), jnp.float32),
        pltpu.VMEM((FD,), jnp.float32),
        pltpu.SemaphoreType.DMA,
        pltpu.SemaphoreType.DMA,
    ],
    compiler_params=_SC_PARAMS,
)
def _sc_fused(x_hbm, lab_hbm, emb_hbm, w_hbm, m_hbm, spart_hbm,
              idx_v, rows0, rows1, cvm, lab_v, sacc_v, sem0, sem1):
    w = lax.axis_index("s") * NC + lax.axis_index("c")
    b0 = w * BPW
    pltpu.sync_copy(x_hbm.at[pl.ds(b0, BPW), :], idx_v)
    pltpu.sync_copy(lab_hbm.at[pl.ds(b0, BPW)], lab_v)
    pltpu.sync_copy(w_hbm, cvm)
    pltpu.sync_copy(m_hbm, sacc_v)

    zeros = jnp.zeros((16,), jnp.float32)

    def cbody(r, carry):
        sl = pl.ds(r * 16, 16)
        cvm[sl] = cvm[sl] * sacc_v[sl]
        return carry
    lax.fori_loop(0, FD // 16, cbody, 0)

    def zbody(r, carry):
        sacc_v[pl.ds(r * 16, 16)] = zeros
        return carry
    lax.fori_loop(0, FD // 16, zbody, 0)

    slots = (rows0, rows1)
    sems = (sem0, sem1)
    nch = BPW // CHB

    def start(g):
        slot, sem = slots[g % 2], sems[g % 2]
        return [pltpu.async_copy(emb_hbm.at[idx_v.at[g * CHB + bb]],
                                 slot.at[bb], sem)
                for bb in range(CHB)]

    pending = start(0)
    inv_b = jnp.float32(1.0 / B)
    for g in range(nch):
        nxt = start(g + 1) if g + 1 < nch else None
        for d_ in pending:
            d_.wait()
        pending = nxt
        slot = slots[g % 2]

        def fbody(f, accs):
            cs = [cvm[pl.ds(f * D + dg * 16, 16)] for dg in range(4)]
            out = []
            for bb in range(CHB):
                a = accs[bb]
                for dg in range(4):
                    a = a + slot[bb, f, pl.ds(dg * 16, 16)] * cs[dg]
                out.append(a)
            return tuple(out)

        accs = lax.fori_loop(0, F, fbody, (zeros,) * CHB)

        lab16 = lab_v[pl.ds((g // 4) * 16, 16)]
        s_sp = []
        for bb in range(CHB):
            l = jnp.sum(accs[bb])
            y = lab16[(g % 4) * CHB + bb]
            lv = jnp.full((16,), l, jnp.float32)
            sig = 1.0 / (1.0 + jnp.exp(-lv))
            s_sp.append((sig - y) * inv_b)

        def f2body(f, carry):
            for dg in range(4):
                sl = pl.ds(f * D + dg * 16, 16)
                a = sacc_v[sl]
                for bb in range(CHB):
                    a = a + slot[bb, f, pl.ds(dg * 16, 16)] * s_sp[bb]
                sacc_v[sl] = a
            return carry
        lax.fori_loop(0, F, f2body, 0)

    pltpu.sync_copy(sacc_v, spart_hbm.at[w])


def _topk_body(spart_ref, w_ref, out_ref):
    s_total = jnp.sum(spart_ref[...], axis=0)
    g = jnp.abs(s_total * w_ref[...])
    total = jnp.sum(g)
    gn = g / total
    gni = lax.bitcast_convert_type(gn, jnp.int32)

    def body(_, carry):
        lo, hi = carry
        mid = lo + (hi - lo) // 2
        cnt = jnp.sum((gni > mid).astype(jnp.int32))
        take = cnt <= KEEP - 1
        return (jnp.where(take, lo, mid + 1), jnp.where(take, mid, hi))

    lo, _ = lax.fori_loop(0, 31, body, (jnp.int32(0), jnp.int32(0x7F800000)))
    out_ref[...] = (gni > lo).astype(jnp.float32)


def _tc_topk(spart2d, wflat):
    return pl.pallas_call(
        _topk_body,
        out_shape=jax.ShapeDtypeStruct((FD,), jnp.float32),
    )(spart2d, wflat)


def kernel(x, labels, emb_weight, model_weight, mask):
    x = x.astype(jnp.int32)
    labels = labels.astype(jnp.float32)
    xpad = jnp.concatenate(
        [x, jnp.zeros((B, FP - F), jnp.int32)], axis=1)
    wflat = model_weight.reshape(FD)
    spart = _sc_fused(xpad, labels, emb_weight, wflat, mask.reshape(FD))
    out = _tc_topk(spart, wflat)
    return out.reshape(F, D)


# speedup vs baseline: 12.2638x; 12.2638x over previous
import functools

import jax
import jax.numpy as jnp
from jax import lax
from jax.experimental import pallas as pl
from jax.experimental.pallas import tpu as pltpu
from jax.experimental.pallas import tpu_sc as plsc

B, F, D, V = 4096, 100, 64, 100000
KEEP = 3200
NC, NS = 2, 16
NW = NC * NS
BPW = B // NW
CHB = 8
FD = F * D

_MESH = plsc.VectorSubcoreMesh(
    core_axis_name="c", subcore_axis_name="s", num_cores=NC, num_subcores=NS)
_SC_PARAMS = pltpu.CompilerParams(
    use_tc_tiling_on_sc=False, needs_layout_passes=False)


@functools.partial(
    pl.kernel,
    out_type=jax.ShapeDtypeStruct((NW, F * 4, 16), jnp.float32),
    mesh=_MESH,
    scratch_types=[
        pltpu.VMEM((BPW, F), jnp.int32),
        pltpu.VMEM((CHB, F, D), jnp.float32),
        pltpu.VMEM((CHB, F, D), jnp.float32),
        pltpu.VMEM((F, D), jnp.float32),
        pltpu.VMEM((BPW,), jnp.float32),
        pltpu.VMEM((F * 4, 16), jnp.float32),
        pltpu.SemaphoreType.DMA,
        pltpu.SemaphoreType.DMA,
    ],
    compiler_params=_SC_PARAMS,
)
def _sc_fused(x_hbm, lab_hbm, emb_hbm, w_hbm, m_hbm, spart_hbm,
              idx_v, rows0, rows1, cvm, lab_v, sacc_v, sem0, sem1):
    w = lax.axis_index("s") * NC + lax.axis_index("c")
    b0 = w * BPW
    pltpu.sync_copy(x_hbm.at[pl.ds(b0, BPW), :], idx_v)
    pltpu.sync_copy(lab_hbm.at[pl.ds(b0, BPW)], lab_v)
    pltpu.sync_copy(w_hbm, cvm)
    pltpu.sync_copy(m_hbm, rows0.at[0])

    zeros = jnp.zeros((16,), jnp.float32)

    def cbody(f, carry):
        for dg in range(4):
            sl = pl.ds(dg * 16, 16)
            cvm[f, sl] = cvm[f, sl] * rows0[0, f, sl]
        return carry
    lax.fori_loop(0, F, cbody, 0)

    def zbody(r, carry):
        sacc_v[r, :] = zeros
        return carry
    lax.fori_loop(0, F * 4, zbody, 0)

    slots = (rows0, rows1)
    sems = (sem0, sem1)
    nch = BPW // CHB

    def start(g):
        slot, sem = slots[g % 2], sems[g % 2]
        return [pltpu.async_copy(emb_hbm.at[idx_v.at[g * CHB + bb]],
                                 slot.at[bb], sem)
                for bb in range(CHB)]

    pending = start(0)
    inv_b = jnp.float32(1.0 / B)
    for g in range(nch):
        nxt = start(g + 1) if g + 1 < nch else None
        for d_ in pending:
            d_.wait()
        pending = nxt
        slot = slots[g % 2]

        def fbody(f, accs):
            cs = [cvm[f, pl.ds(dg * 16, 16)] for dg in range(4)]
            out = []
            for bb in range(CHB):
                a = accs[bb]
                for dg in range(4):
                    a = a + slot[bb, f, pl.ds(dg * 16, 16)] * cs[dg]
                out.append(a)
            return tuple(out)

        accs = lax.fori_loop(0, F, fbody, (zeros,) * CHB, unroll=2)

        lab16 = lab_v[pl.ds((g // 2) * 16, 16)]
        s_sp = []
        for bb in range(CHB):
            l = jnp.sum(accs[bb])
            y = lab16[(g % 2) * CHB + bb]
            lv = jnp.full((16,), l, jnp.float32)
            sig = 1.0 / (1.0 + jnp.exp(-lv))
            s_sp.append((sig - y) * inv_b)

        def f2body(f, carry):
            for dg in range(4):
                r = f * 4 + dg
                a = sacc_v[r, :]
                for bb in range(CHB):
                    a = a + slot[bb, f, pl.ds(dg * 16, 16)] * s_sp[bb]
                sacc_v[r, :] = a
            return carry
        lax.fori_loop(0, F, f2body, 0, unroll=2)

    pltpu.sync_copy(sacc_v, spart_hbm.at[w])


def _topk_body(spart_ref, w_ref, out_ref):
    s_total = jnp.sum(spart_ref[...], axis=0)
    g = jnp.abs(s_total * w_ref[...])
    total = jnp.sum(g)
    gn = g / total
    gni = lax.bitcast_convert_type(gn, jnp.int32)

    def body(_, carry):
        lo, hi = carry
        mid = lo + (hi - lo) // 2
        cnt = jnp.sum((gni > mid).astype(jnp.int32))
        take = cnt <= KEEP - 1
        return (jnp.where(take, lo, mid + 1), jnp.where(take, mid, hi))

    lo, _ = lax.fori_loop(0, 31, body, (jnp.int32(0), jnp.int32(0x7F800000)))
    out_ref[...] = (gni > lo).astype(jnp.float32)


def _tc_topk(spart2d, wflat):
    return pl.pallas_call(
        _topk_body,
        out_shape=jax.ShapeDtypeStruct((FD,), jnp.float32),
    )(spart2d, wflat)


def kernel(x, labels, emb_weight, model_weight, mask):
    x = x.astype(jnp.int32)
    labels = labels.astype(jnp.float32)
    spart = _sc_fused(x, labels, emb_weight, model_weight, mask)
    out = _tc_topk(spart.reshape(NW, FD), model_weight.reshape(FD))
    return out.reshape(F, D)


# speedup vs baseline: 12.4775x; 1.0174x over previous
import functools

import jax
import jax.numpy as jnp
from jax import lax
from jax.experimental import pallas as pl
from jax.experimental.pallas import tpu as pltpu
from jax.experimental.pallas import tpu_sc as plsc

B, F, D, V = 4096, 100, 64, 100000
KEEP = 3200
NC, NS = 2, 16
NW = NC * NS
BPW = B // NW
CHB = 8
FD = F * D

_MESH = plsc.VectorSubcoreMesh(
    core_axis_name="c", subcore_axis_name="s", num_cores=NC, num_subcores=NS)
_SC_PARAMS = pltpu.CompilerParams(
    use_tc_tiling_on_sc=False, needs_layout_passes=False)


@functools.partial(
    pl.kernel,
    out_type=jax.ShapeDtypeStruct((NW, F * 4, 16), jnp.float32),
    mesh=_MESH,
    scratch_types=[
        pltpu.VMEM((BPW, F), jnp.int32),
        pltpu.VMEM((CHB, F, D), jnp.float32),
        pltpu.VMEM((CHB, F, D), jnp.float32),
        pltpu.VMEM((F, D), jnp.float32),
        pltpu.VMEM((BPW,), jnp.float32),
        pltpu.VMEM((F * 4, 16), jnp.float32),
        pltpu.SemaphoreType.DMA,
        pltpu.SemaphoreType.DMA,
    ],
    compiler_params=_SC_PARAMS,
)
def _sc_fused(x_hbm, lab_hbm, emb_hbm, w_hbm, m_hbm, spart_hbm,
              idx_v, rows0, rows1, cvm, lab_v, sacc_v, sem0, sem1):
    w = lax.axis_index("s") * NC + lax.axis_index("c")
    b0 = w * BPW
    pltpu.sync_copy(x_hbm.at[pl.ds(b0, BPW), :], idx_v)
    pltpu.sync_copy(lab_hbm.at[pl.ds(b0, BPW)], lab_v)
    pltpu.sync_copy(w_hbm, cvm)
    pltpu.sync_copy(m_hbm, rows0.at[0])

    zeros = jnp.zeros((16,), jnp.float32)

    def cbody(f, carry):
        for dg in range(4):
            sl = pl.ds(dg * 16, 16)
            cvm[f, sl] = cvm[f, sl] * rows0[0, f, sl]
        return carry
    lax.fori_loop(0, F, cbody, 0)

    def zbody(r, carry):
        sacc_v[r, :] = zeros
        return carry
    lax.fori_loop(0, F * 4, zbody, 0)

    slots = (rows0, rows1)
    sems = (sem0, sem1)
    nch = BPW // CHB

    def start(g):
        slot, sem = slots[g % 2], sems[g % 2]
        return [pltpu.async_copy(emb_hbm.at[idx_v.at[g * CHB + bb]],
                                 slot.at[bb], sem)
                for bb in range(CHB)]

    pending = start(0)
    inv_b = jnp.float32(1.0 / B)
    for g in range(nch):
        nxt = start(g + 1) if g + 1 < nch else None
        for d_ in pending:
            d_.wait()
        pending = nxt
        slot = slots[g % 2]

        def fbody(f, accs):
            cs = [cvm[f, pl.ds(dg * 16, 16)] for dg in range(4)]
            out = []
            for bb in range(CHB):
                a = accs[bb]
                for dg in range(4):
                    a = a + slot[bb, f, pl.ds(dg * 16, 16)] * cs[dg]
                out.append(a)
            return tuple(out)

        accs = lax.fori_loop(0, F, fbody, (zeros,) * CHB)

        lab16 = lab_v[pl.ds((g // 2) * 16, 16)]
        s_sp = []
        for bb in range(CHB):
            l = jnp.sum(accs[bb])
            y = lab16[(g % 2) * CHB + bb]
            lv = jnp.full((16,), l, jnp.float32)
            sig = 1.0 / (1.0 + jnp.exp(-lv))
            s_sp.append((sig - y) * inv_b)

        def f2body(f, carry):
            for dg in range(4):
                r = f * 4 + dg
                a = sacc_v[r, :]
                for bb in range(CHB):
                    a = a + slot[bb, f, pl.ds(dg * 16, 16)] * s_sp[bb]
                sacc_v[r, :] = a
            return carry
        lax.fori_loop(0, F, f2body, 0)

    pltpu.sync_copy(sacc_v, spart_hbm.at[w])


def _topk_body(spart_ref, w_ref, out_ref):
    s_total = jnp.sum(spart_ref[...], axis=0)
    g = jnp.abs(s_total * w_ref[...])
    total = jnp.sum(g)
    gn = g / total
    gni = lax.bitcast_convert_type(gn, jnp.int32)

    def body(_, carry):
        lo, hi = carry
        mid = lo + (hi - lo) // 2
        cnt = jnp.sum((gni > mid).astype(jnp.int32))
        take = cnt <= KEEP - 1
        return (jnp.where(take, lo, mid + 1), jnp.where(take, mid, hi))

    lo, _ = lax.fori_loop(0, 31, body, (jnp.int32(0), jnp.int32(0x7F800000)))
    out_ref[...] = (gni > lo).astype(jnp.float32)


def _tc_topk(spart2d, wflat):
    return pl.pallas_call(
        _topk_body,
        out_shape=jax.ShapeDtypeStruct((FD,), jnp.float32),
    )(spart2d, wflat)


def kernel(x, labels, emb_weight, model_weight, mask):
    x = x.astype(jnp.int32)
    labels = labels.astype(jnp.float32)
    spart = _sc_fused(x, labels, emb_weight, model_weight, mask)
    out = _tc_topk(spart.reshape(NW, FD), model_weight.reshape(FD))
    return out.reshape(F, D)
